# Initial kernel scaffold; baseline (speedup 1.0000x reference)
import jax
import jax.numpy as jnp
from jax.experimental import pallas as pl


def kernel(x, edge_index, W1, b1, W2, b2):
    raise NotImplementedError("write your pallas kernel here")


# speedup vs baseline: 7.4958x; 7.4958x over previous
import functools

import jax
import jax.numpy as jnp
from jax import lax
from jax.experimental import pallas as pl
from jax.experimental.pallas import tpu as pltpu
from jax.experimental.pallas import tpu_sc as plsc

N_NODES = 10000
D = 128
DH = D // 2
N_EDGES = 320000

NC = 2
NS = 16
EPT = N_EDGES // NS
CHUNK = 80
NCHUNK = EPT // CHUNK
ROWS_PER_TILE = 624
TAIL_ROWS = N_NODES - NS * ROWS_PER_TILE
TAIL_OFF = NS * ROWS_PER_TILE

_mesh = plsc.VectorSubcoreMesh(core_axis_name="c", subcore_axis_name="s",
                               num_cores=NC, num_subcores=NS)


@functools.partial(
    pl.kernel,
    out_type=jax.ShapeDtypeStruct((NC, N_NODES, DH), jnp.float32),
    mesh=_mesh,
    scratch_types=[
        pltpu.VMEM((NCHUNK, CHUNK), jnp.int32),
        pltpu.VMEM((NCHUNK, CHUNK), jnp.int32),
        pltpu.VMEM((CHUNK, DH), jnp.float32),
        pltpu.VMEM((CHUNK, DH), jnp.float32),
        pltpu.VMEM_SHARED((N_NODES, DH), jnp.float32),
        pltpu.SemaphoreType.DMA,
        pltpu.SemaphoreType.DMA,
    ],
    compiler_params=pltpu.CompilerParams(use_tc_tiling_on_sc=False),
)
def _sc_aggregate(xt_hbm, src_hbm, dst_hbm, out_hbm,
                  src_v, dst_v, rows_a, rows_b, agg_sh, sem_a, sem_b):
    c = lax.axis_index("c")
    s = lax.axis_index("s")
    x_half = xt_hbm.at[c]

    r0 = s * ROWS_PER_TILE
    pltpu.sync_copy(x_half.at[pl.ds(r0, ROWS_PER_TILE)],
                    agg_sh.at[pl.ds(r0, ROWS_PER_TILE)])

    @pl.when(s == NS - 1)
    def _():
        pltpu.sync_copy(x_half.at[pl.ds(TAIL_OFF, TAIL_ROWS)],
                        agg_sh.at[pl.ds(TAIL_OFF, TAIL_ROWS)])

    pltpu.sync_copy(src_hbm.at[s], src_v)
    pltpu.sync_copy(dst_hbm.at[s], dst_v)
    plsc.subcore_barrier()

    def two_chunks(it, carry):
        j0 = 2 * it
        j1 = j0 + 1
        da = pltpu.async_copy(x_half.at[src_v.at[j0]], rows_a, sem_a)
        db = pltpu.async_copy(x_half.at[src_v.at[j1]], rows_b, sem_b)
        da.wait()
        pltpu.sync_copy(rows_a, agg_sh.at[dst_v.at[j0]], add=True)
        db.wait()
        pltpu.sync_copy(rows_b, agg_sh.at[dst_v.at[j1]], add=True)
        return carry

    lax.fori_loop(0, NCHUNK // 2, two_chunks, 0)

    plsc.subcore_barrier()

    out_half = out_hbm.at[c]
    pltpu.sync_copy(agg_sh.at[pl.ds(r0, ROWS_PER_TILE)],
                    out_half.at[pl.ds(r0, ROWS_PER_TILE)])

    @pl.when(s == NS - 1)
    def _():
        pltpu.sync_copy(agg_sh.at[pl.ds(TAIL_OFF, TAIL_ROWS)],
                        out_half.at[pl.ds(TAIL_OFF, TAIL_ROWS)])


BLK = 1000


def _mlp_body(p_ref, w1_ref, b1_ref, w2_ref, b2_ref, o_ref):
    h = jnp.concatenate([p_ref[0], p_ref[1]], axis=-1)
    h = jnp.dot(h, w1_ref[...], preferred_element_type=jnp.float32) + b1_ref[...]
    h = jnp.maximum(h, 0.0)
    o_ref[...] = jnp.dot(h, w2_ref[...], preferred_element_type=jnp.float32) + b2_ref[...]


_mlp = pl.pallas_call(
    _mlp_body,
    grid=(N_NODES // BLK,),
    in_specs=[
        pl.BlockSpec((NC, BLK, DH), lambda i: (0, i, 0)),
        pl.BlockSpec((D, D), lambda i: (0, 0)),
        pl.BlockSpec((1, D), lambda i: (0, 0)),
        pl.BlockSpec((D, D), lambda i: (0, 0)),
        pl.BlockSpec((1, D), lambda i: (0, 0)),
    ],
    out_specs=pl.BlockSpec((BLK, D), lambda i: (i, 0)),
    out_shape=jax.ShapeDtypeStruct((N_NODES, D), jnp.float32),
)


def kernel(x, edge_index, W1, b1, W2, b2):
    xt = x.reshape(N_NODES, NC, DH).transpose(1, 0, 2)
    src = edge_index[0].astype(jnp.int32).reshape(NS, NCHUNK, CHUNK)
    dst = edge_index[1].astype(jnp.int32).reshape(NS, NCHUNK, CHUNK)
    p = _sc_aggregate(xt, src, dst)
    return _mlp(p, W1, b1.reshape(1, D), W2, b2.reshape(1, D))


# speedup vs baseline: 11.2602x; 1.5022x over previous
import functools

import jax
import jax.numpy as jnp
from jax import lax
from jax.experimental import pallas as pl
from jax.experimental.pallas import tpu as pltpu
from jax.experimental.pallas import tpu_sc as plsc

N_NODES = 10000
D = 128
DH = D // 2
N_EDGES = 320000

NC = 2
NS = 16
EPT = N_EDGES // NS
CHUNK = 200
NG = EPT // CHUNK
ROWS_PER_TILE = 624
TAIL_ROWS = N_NODES - NS * ROWS_PER_TILE
TAIL_OFF = NS * ROWS_PER_TILE

_mesh = plsc.VectorSubcoreMesh(core_axis_name="c", subcore_axis_name="s",
                               num_cores=NC, num_subcores=NS)


@functools.partial(
    pl.kernel,
    out_type=jax.ShapeDtypeStruct((NC, N_NODES, DH), jnp.float32),
    mesh=_mesh,
    scratch_types=[
        pltpu.VMEM((NG, CHUNK), jnp.int32),
        pltpu.VMEM((NG, CHUNK), jnp.int32),
        pltpu.VMEM((CHUNK, DH), jnp.float32),
        pltpu.VMEM((CHUNK, DH), jnp.float32),
        pltpu.VMEM_SHARED((N_NODES, DH), jnp.float32),
        pltpu.SemaphoreType.DMA,
        pltpu.SemaphoreType.DMA,
    ],
    compiler_params=pltpu.CompilerParams(use_tc_tiling_on_sc=False),
)
def _sc_aggregate(xt_hbm, src_hbm, dst_hbm, out_hbm,
                  src_v, dst_v, rows_a, rows_b, agg_sh, sem_a, sem_b):
    c = lax.axis_index("c")
    s = lax.axis_index("s")
    x_half = xt_hbm.at[c]

    r0 = s * ROWS_PER_TILE
    pltpu.sync_copy(x_half.at[pl.ds(r0, ROWS_PER_TILE)],
                    agg_sh.at[pl.ds(r0, ROWS_PER_TILE)])

    @pl.when(s == NS - 1)
    def _():
        pltpu.sync_copy(x_half.at[pl.ds(TAIL_OFF, TAIL_ROWS)],
                        agg_sh.at[pl.ds(TAIL_OFF, TAIL_ROWS)])

    pltpu.sync_copy(src_hbm.at[s], src_v)
    pltpu.sync_copy(dst_hbm.at[s], dst_v)
    plsc.subcore_barrier()

    pltpu.async_copy(x_half.at[src_v.at[0]], rows_a, sem_a)

    def two_groups(t, carry):
        g0 = 2 * t
        g1 = g0 + 1
        pltpu.async_copy(x_half.at[src_v.at[g1]], rows_b, sem_b)
        pltpu.make_async_copy(x_half.at[src_v.at[0]], rows_a, sem_a).wait()
        pltpu.sync_copy(rows_a, agg_sh.at[dst_v.at[g0]], add=True)

        @pl.when(t < NG // 2 - 1)
        def _():
            pltpu.async_copy(x_half.at[src_v.at[g0 + 2]], rows_a, sem_a)

        pltpu.make_async_copy(x_half.at[src_v.at[0]], rows_b, sem_b).wait()
        pltpu.sync_copy(rows_b, agg_sh.at[dst_v.at[g1]], add=True)
        return carry

    lax.fori_loop(0, NG // 2, two_groups, 0)

    plsc.subcore_barrier()

    out_half = out_hbm.at[c]
    pltpu.sync_copy(agg_sh.at[pl.ds(r0, ROWS_PER_TILE)],
                    out_half.at[pl.ds(r0, ROWS_PER_TILE)])

    @pl.when(s == NS - 1)
    def _():
        pltpu.sync_copy(agg_sh.at[pl.ds(TAIL_OFF, TAIL_ROWS)],
                        out_half.at[pl.ds(TAIL_OFF, TAIL_ROWS)])


BLK = 1000


def _mlp_body(p_ref, w1_ref, b1_ref, w2_ref, b2_ref, o_ref):
    h = jnp.concatenate([p_ref[0], p_ref[1]], axis=-1)
    h = jnp.dot(h, w1_ref[...], preferred_element_type=jnp.float32) + b1_ref[...]
    h = jnp.maximum(h, 0.0)
    o_ref[...] = jnp.dot(h, w2_ref[...], preferred_element_type=jnp.float32) + b2_ref[...]


_mlp = pl.pallas_call(
    _mlp_body,
    grid=(N_NODES // BLK,),
    in_specs=[
        pl.BlockSpec((NC, BLK, DH), lambda i: (0, i, 0)),
        pl.BlockSpec((D, D), lambda i: (0, 0)),
        pl.BlockSpec((1, D), lambda i: (0, 0)),
        pl.BlockSpec((D, D), lambda i: (0, 0)),
        pl.BlockSpec((1, D), lambda i: (0, 0)),
    ],
    out_specs=pl.BlockSpec((BLK, D), lambda i: (i, 0)),
    out_shape=jax.ShapeDtypeStruct((N_NODES, D), jnp.float32),
)


def kernel(x, edge_index, W1, b1, W2, b2):
    xt = x.reshape(N_NODES, NC, DH).transpose(1, 0, 2)
    src = edge_index[0].astype(jnp.int32).reshape(NS, NG, CHUNK)
    dst = edge_index[1].astype(jnp.int32).reshape(NS, NG, CHUNK)
    p = _sc_aggregate(xt, src, dst)
    return _mlp(p, W1, b1.reshape(1, D), W2, b2.reshape(1, D))


# speedup vs baseline: 11.2826x; 1.0020x over previous
import functools

import jax
import jax.numpy as jnp
from jax import lax
from jax.experimental import pallas as pl
from jax.experimental.pallas import tpu as pltpu
from jax.experimental.pallas import tpu_sc as plsc

N_NODES = 10000
D = 128
DH = D // 2
N_EDGES = 320000

NC = 2
NS = 16
EPT = N_EDGES // NS
CHUNK = 400
NG = EPT // CHUNK
HG = NG // 2
ROWS_PER_TILE = 624
TAIL_ROWS = N_NODES - NS * ROWS_PER_TILE
TAIL_OFF = NS * ROWS_PER_TILE

_mesh = plsc.VectorSubcoreMesh(core_axis_name="c", subcore_axis_name="s",
                               num_cores=NC, num_subcores=NS)


@functools.partial(
    pl.kernel,
    out_type=jax.ShapeDtypeStruct((NC, N_NODES, DH), jnp.float32),
    mesh=_mesh,
    scratch_types=[
        pltpu.VMEM((HG, CHUNK), jnp.int32),
        pltpu.VMEM((HG, CHUNK), jnp.int32),
        pltpu.VMEM((CHUNK, DH), jnp.float32),
        pltpu.VMEM((CHUNK, DH), jnp.float32),
        pltpu.VMEM_SHARED((N_NODES, DH), jnp.float32),
        pltpu.SemaphoreType.DMA,
        pltpu.SemaphoreType.DMA,
    ],
    compiler_params=pltpu.CompilerParams(use_tc_tiling_on_sc=False),
)
def _sc_aggregate(xt_hbm, src_hbm, dst_hbm, out_hbm,
                  src_v, dst_v, rows_a, rows_b, agg_sh, sem_a, sem_b):
    c = lax.axis_index("c")
    s = lax.axis_index("s")
    x_half = xt_hbm.at[c]

    r0 = s * ROWS_PER_TILE
    pltpu.sync_copy(x_half.at[pl.ds(r0, ROWS_PER_TILE)],
                    agg_sh.at[pl.ds(r0, ROWS_PER_TILE)])

    @pl.when(s == NS - 1)
    def _():
        pltpu.sync_copy(x_half.at[pl.ds(TAIL_OFF, TAIL_ROWS)],
                        agg_sh.at[pl.ds(TAIL_OFF, TAIL_ROWS)])

    plsc.subcore_barrier()

    for h in (0, 1):
        pltpu.sync_copy(src_hbm.at[s].at[pl.ds(h * HG, HG)], src_v)
        pltpu.sync_copy(dst_hbm.at[s].at[pl.ds(h * HG, HG)], dst_v)
        pltpu.async_copy(x_half.at[src_v.at[0]], rows_a, sem_a)

        def two_groups(t, carry):
            l0 = 2 * t
            l1 = l0 + 1
            pltpu.async_copy(x_half.at[src_v.at[l1]], rows_b, sem_b)
            pltpu.make_async_copy(x_half.at[src_v.at[0]], rows_a, sem_a).wait()
            pltpu.sync_copy(rows_a, agg_sh.at[dst_v.at[l0]], add=True)
            pltpu.async_copy(x_half.at[src_v.at[l0 + 2]], rows_a, sem_a)
            pltpu.make_async_copy(x_half.at[src_v.at[0]], rows_b, sem_b).wait()
            pltpu.sync_copy(rows_b, agg_sh.at[dst_v.at[l1]], add=True)
            return carry

        lax.fori_loop(0, HG // 2, two_groups, 0)
        pltpu.make_async_copy(x_half.at[src_v.at[0]], rows_a, sem_a).wait()
        pltpu.sync_copy(rows_a, agg_sh.at[dst_v.at[HG - 1]], add=True)

    plsc.subcore_barrier()

    out_half = out_hbm.at[c]
    pltpu.sync_copy(agg_sh.at[pl.ds(r0, ROWS_PER_TILE)],
                    out_half.at[pl.ds(r0, ROWS_PER_TILE)])

    @pl.when(s == NS - 1)
    def _():
        pltpu.sync_copy(agg_sh.at[pl.ds(TAIL_OFF, TAIL_ROWS)],
                        out_half.at[pl.ds(TAIL_OFF, TAIL_ROWS)])


BLK = 1000


def _mlp_body(p_ref, w1_ref, b1_ref, w2_ref, b2_ref, o_ref):
    h = jnp.concatenate([p_ref[0], p_ref[1]], axis=-1)
    h = jnp.dot(h, w1_ref[...], preferred_element_type=jnp.float32) + b1_ref[...]
    h = jnp.maximum(h, 0.0)
    o_ref[...] = jnp.dot(h, w2_ref[...], preferred_element_type=jnp.float32) + b2_ref[...]


_mlp = pl.pallas_call(
    _mlp_body,
    grid=(N_NODES // BLK,),
    in_specs=[
        pl.BlockSpec((NC, BLK, DH), lambda i: (0, i, 0)),
        pl.BlockSpec((D, D), lambda i: (0, 0)),
        pl.BlockSpec((1, D), lambda i: (0, 0)),
        pl.BlockSpec((D, D), lambda i: (0, 0)),
        pl.BlockSpec((1, D), lambda i: (0, 0)),
    ],
    out_specs=pl.BlockSpec((BLK, D), lambda i: (i, 0)),
    out_shape=jax.ShapeDtypeStruct((N_NODES, D), jnp.float32),
)


def kernel(x, edge_index, W1, b1, W2, b2):
    xt = x.reshape(N_NODES, NC, DH).transpose(1, 0, 2)
    src = edge_index[0].astype(jnp.int32).reshape(NS, NG, CHUNK)
    dst = edge_index[1].astype(jnp.int32).reshape(NS, NG, CHUNK)
    p = _sc_aggregate(xt, src, dst)
    return _mlp(p, W1, b1.reshape(1, D), W2, b2.reshape(1, D))
